# baseline (speedup 1.0000x reference)
import jax
import jax.numpy as jnp
from jax.experimental import pallas as pl
from jax.experimental.pallas import tpu as pltpu

TAU = 0.5
NS = 4
D = 768
BLK = 2048


def _body(s1_ref, s2_ref, u1_ref, u2_ref, w_ref, b_ref,
          l1_ref, l2_ref, y1_ref, y2_ref, r1_ref, r2_ref, sel_ref):
    w = w_ref[...]

    def route(s, u, l_ref, y_ref, r_ref):
        lt = jax.lax.dot_general(
            w, s, (((1,), (1,)), ((), ())),
            preferred_element_type=jnp.float32) + b_ref[...]
        l_ref[...] = lt
        g = (lt + -jnp.log(-jnp.log(u))) / TAU
        m = jnp.max(g, axis=0, keepdims=True)
        e = jnp.exp(g - m)
        y = e / jnp.sum(e, axis=0, keepdims=True)
        y_ref[...] = y
        idx = jnp.argmax(y, axis=0)
        oh = (jax.lax.broadcasted_iota(jnp.int32, (NS, BLK), 0)
              == idx[None, :]).astype(jnp.float32)
        r_ref[...] = (oh - y) + y

    route(s1_ref[...], u1_ref[...], l1_ref, y1_ref, r1_ref)
    route(s2_ref[...], u2_ref[...], l2_ref, y2_ref, r2_ref)

    @pl.when(pl.program_id(0) == 0)
    def _():
        s0 = r1_ref[0, 0]
        s1 = r1_ref[1, 0]
        s2 = r1_ref[2, 0]
        s3 = r1_ref[3, 0]
        bi = jnp.int32(0)
        bv = s0
        bi = jnp.where(s1 > bv, jnp.int32(1), bi)
        bv = jnp.maximum(bv, s1)
        bi = jnp.where(s2 > bv, jnp.int32(2), bi)
        bv = jnp.maximum(bv, s2)
        bi = jnp.where(s3 > bv, jnp.int32(3), bi)
        sel_ref[0, 0] = bi


def kernel(S_t, u_noise, W, b):
    n = S_t.shape[0]
    h = n // 2
    grid = (h // BLK,)
    u_t = u_noise.T
    b2 = b.reshape(NS, 1)
    big = jax.ShapeDtypeStruct((NS, h), jnp.float32)

    l1, l2, y1, y2, r1, r2, sel = pl.pallas_call(
        _body,
        grid=grid,
        in_specs=[
            pl.BlockSpec((BLK, D), lambda i: (i, 0)),
            pl.BlockSpec((BLK, D), lambda i, _h=h // BLK: (i + _h, 0)),
            pl.BlockSpec((NS, BLK), lambda i: (0, i)),
            pl.BlockSpec((NS, BLK), lambda i, _h=h // BLK: (0, i + _h)),
            pl.BlockSpec((NS, D), lambda i: (0, 0)),
            pl.BlockSpec((NS, 1), lambda i: (0, 0)),
        ],
        out_specs=[
            pl.BlockSpec((NS, BLK), lambda i: (0, i)),
            pl.BlockSpec((NS, BLK), lambda i: (0, i)),
            pl.BlockSpec((NS, BLK), lambda i: (0, i)),
            pl.BlockSpec((NS, BLK), lambda i: (0, i)),
            pl.BlockSpec((NS, BLK), lambda i: (0, i)),
            pl.BlockSpec((NS, BLK), lambda i: (0, i)),
            pl.BlockSpec((1, 1), lambda i: (0, 0), memory_space=pltpu.SMEM),
        ],
        out_shape=[big, big, big, big, big, big,
                   jax.ShapeDtypeStruct((1, 1), jnp.int32)],
    )(S_t, S_t, u_t, u_t, W, b2)

    rp = jnp.concatenate([r1, r2], axis=1).T
    lo = jnp.concatenate([l1, l2], axis=1).T
    ys = jnp.concatenate([y1, y2], axis=1).T
    return (rp, sel.reshape(()), lo, ys)
